# Initial kernel scaffold; baseline (speedup 1.0000x reference)
import jax
import jax.numpy as jnp
from jax.experimental import pallas as pl


def kernel(logits):
    raise NotImplementedError("write your pallas kernel here")


# speedup vs baseline: 45.7813x; 45.7813x over previous
import functools

import numpy as np

import jax
import jax.numpy as jnp
from jax import lax
from jax.experimental import pallas as pl
from jax.experimental.pallas import tpu as pltpu
from jax.experimental.pallas import tpu_sc as plsc

ROWS = 64
COLS = 32768
L = 16
NV = COLS // L
NB = 1024
BSHIFT = 32 - 10
CAND = 1024
KSEL = 64
INT_MIN = np.int32(-(2 ** 31))
INT_MAXPOS = np.int32(0x7FFFFFFF)


def _skey(v_f32):
    s = lax.bitcast_convert_type(v_f32, jnp.int32)
    return s ^ ((s >> 31) & INT_MAXPOS)


def _row_topk(a_v, n_v, sk_v, hist_v, ck_v, ci_v):
    lanes = lax.iota(jnp.int32, L)
    zeros_i = jnp.zeros((L,), jnp.int32)
    ones_i = jnp.full((L,), 1, jnp.int32)
    ones_f = jnp.full((L,), 1.0, jnp.float32)

    def clear_body(i, _):
        hist_v[pl.ds(i * L, L)] = zeros_i
        return 0

    lax.fori_loop(0, NB * L // L, clear_body, 0)

    def p1_body(i, _):
        g = a_v[pl.ds(i * L, L)] + n_v[pl.ds(i * L, L)]
        sk = _skey(g)
        sk_v[pl.ds(i * L, L)] = sk
        bucket = lax.shift_right_logical(sk ^ INT_MIN, BSHIFT)
        plsc.addupdate_scatter(hist_v, [bucket * L + lanes], ones_i)
        return 0

    lax.fori_loop(0, NV, p1_body, 0)

    def scan_cond(st):
        _, _, found = st
        return jnp.logical_not(found)

    def scan_body(st):
        b, cum, _ = st
        cnt_b = jnp.sum(hist_v[pl.ds(b * L, L)])
        found = cum + cnt_b >= KSEL
        return (jnp.where(found, b, b - 1), jnp.where(found, cum, cum + cnt_b), found)

    b_star, _, _ = lax.while_loop(
        scan_cond, scan_body, (jnp.int32(NB - 1), jnp.int32(0), False))

    slo = (b_star << BSHIFT) ^ INT_MIN

    def col_body(i, cnt):
        sk = sk_v[pl.ds(i * L, L)]
        m = sk >= slo
        mi = jnp.where(m, ones_i, zeros_i)
        pos = cnt + plsc.cumsum(mi) - 1
        m = jnp.logical_and(m, pos < CAND)
        plsc.store_scatter(ck_v, [pos], sk, mask=m)
        plsc.store_scatter(ci_v, [pos], i * L + lanes, mask=m)
        return cnt + jnp.sum(mi)

    cnt = lax.fori_loop(0, NV, col_body, jnp.int32(0))
    n_cand = jnp.minimum(cnt, jnp.int32(CAND))
    plsc.store_scatter(ck_v, [jnp.minimum(n_cand + lanes, jnp.int32(CAND + L - 1))],
                       jnp.full((L,), INT_MIN, jnp.int32))
    nvw = (n_cand + (L - 1)) // L

    def count_gt(x):
        def body(j, acc):
            v = ck_v[pl.ds(j * L, L)]
            return acc + jnp.sum(jnp.where(v > x, ones_i, zeros_i))
        return lax.fori_loop(0, nvw, body, jnp.int32(0))

    def bis_body(_, st):
        lo, hi = st
        mid = lo + (hi - lo + 1) // 2
        c = count_gt(mid)
        take_hi = c < KSEL
        return (jnp.where(take_hi, lo, mid), jnp.where(take_hi, mid, hi))

    lo0 = slo - 1
    hi0 = slo + jnp.int32((1 << BSHIFT) - 1)
    _, tkey = lax.fori_loop(0, BSHIFT + 1, bis_body, (lo0, hi0))

    n_gt = count_gt(tkey)
    need = jnp.int32(KSEL) - n_gt

    def out_body(i, _):
        sk = sk_v[pl.ds(i * L, L)]
        a_v[pl.ds(i * L, L)] = jnp.where(sk > tkey, ones_f, 0.0)
        return 0

    lax.fori_loop(0, NV, out_body, 0)

    def eq_body(j, ecnt):
        v = ck_v[pl.ds(j * L, L)]
        pv = ci_v[pl.ds(j * L, L)]
        em = v == tkey
        emi = jnp.where(em, ones_i, zeros_i)
        pref = ecnt + plsc.cumsum(emi)
        sel = jnp.logical_and(em, pref <= need)
        plsc.store_scatter(a_v, [pv], ones_f, mask=sel)
        return ecnt + jnp.sum(emi)

    lax.fori_loop(0, nvw, eq_body, jnp.int32(0))


def _make_sc_kernel():
    nc, ns = 2, 16
    mesh = plsc.VectorSubcoreMesh(
        core_axis_name="c", subcore_axis_name="s", num_cores=nc, num_subcores=ns)
    rows_per_w = ROWS // (nc * ns)

    @functools.partial(
        pl.kernel,
        out_type=jax.ShapeDtypeStruct((ROWS, COLS), jnp.float32),
        mesh=mesh,
        compiler_params=pltpu.CompilerParams(needs_layout_passes=False),
        scratch_types=[
            pltpu.VMEM((COLS,), jnp.float32),
            pltpu.VMEM((COLS,), jnp.float32),
            pltpu.VMEM((COLS,), jnp.int32),
            pltpu.VMEM((NB * L,), jnp.int32),
            pltpu.VMEM((CAND + L,), jnp.int32),
            pltpu.VMEM((CAND + L,), jnp.int32),
        ],
    )
    def k(logits_hbm, noise_hbm, out_hbm, a_v, n_v, sk_v, hist_v, ck_v, ci_v):
        wid = lax.axis_index("s") * nc + lax.axis_index("c")
        for rr in range(rows_per_w):
            row = wid * rows_per_w + rr
            pltpu.sync_copy(logits_hbm.at[row], a_v)
            pltpu.sync_copy(noise_hbm.at[row], n_v)
            _row_topk(a_v, n_v, sk_v, hist_v, ck_v, ci_v)
            pltpu.sync_copy(a_v, out_hbm.at[row])

    return k


_sc_kernel = None


def kernel(logits):
    global _sc_kernel
    if _sc_kernel is None:
        _sc_kernel = _make_sc_kernel()
    gkey = jax.random.fold_in(jax.random.key(0), 1)
    noise = jax.random.gumbel(gkey, (ROWS, COLS), jnp.float32)
    return _sc_kernel(logits, noise)
